# baseline (speedup 1.0000x reference)
import functools

import jax
import jax.numpy as jnp
from jax import lax
from jax.experimental import pallas as pl
from jax.experimental.pallas import tpu as pltpu
from jax.experimental.pallas import tpu_sc as plsc

_NC = 2
_NS = 16
_L = 16
_NW = _NC * _NS


def _sc_gather_coeffs(t, table_a, table_b):
    B = t.shape[0]
    T = table_a.shape[0]
    per_w = B // _NW

    mesh = plsc.VectorSubcoreMesh(core_axis_name="c", subcore_axis_name="s")

    @functools.partial(
        pl.kernel,
        out_type=[
            jax.ShapeDtypeStruct((B,), jnp.float32),
            jax.ShapeDtypeStruct((B,), jnp.float32),
        ],
        mesh=mesh,
        compiler_params=pltpu.CompilerParams(needs_layout_passes=False),
        scratch_types=[
            pltpu.VMEM((per_w,), jnp.int32),
            pltpu.VMEM((T,), jnp.float32),
            pltpu.VMEM((T,), jnp.float32),
            pltpu.VMEM((per_w,), jnp.float32),
            pltpu.VMEM((per_w,), jnp.float32),
        ],
    )
    def gather_kernel(t_hbm, a_hbm, b_hbm, out_a_hbm, out_b_hbm,
                      idx_v, a_v, b_v, oa_v, ob_v):
        wid = lax.axis_index("s") * _NC + lax.axis_index("c")
        base = wid * per_w
        pltpu.sync_copy(t_hbm.at[pl.ds(base, per_w)], idx_v)
        pltpu.sync_copy(a_hbm, a_v)
        pltpu.sync_copy(b_hbm, b_v)
        for i in range(per_w // _L):
            iv = idx_v[pl.ds(i * _L, _L)]
            oa_v[pl.ds(i * _L, _L)] = plsc.load_gather(a_v, [iv])
            ob_v[pl.ds(i * _L, _L)] = plsc.load_gather(b_v, [iv])
        pltpu.sync_copy(oa_v, out_a_hbm.at[pl.ds(base, per_w)])
        pltpu.sync_copy(ob_v, out_b_hbm.at[pl.ds(base, per_w)])

    return gather_kernel(t, table_a, table_b)


def _scale_body(a_ref, b_ref, x_ref, n_ref, xt_ref, no_ref):
    n = n_ref[...]
    xt_ref[...] = a_ref[...] * x_ref[...] + b_ref[...] * n
    no_ref[...] = n


def kernel(x_0, t, sqrt_alphas_cumprod, sqrt_one_minus_alphas_cumprod, noise):
    B, C, L = x_0.shape
    F = C * L
    BB = 16

    coeff_a, coeff_b = _sc_gather_coeffs(
        t, sqrt_alphas_cumprod, sqrt_one_minus_alphas_cumprod)

    x2 = x_0.reshape(B, F)
    n2 = noise.reshape(B, F)
    ca = coeff_a.reshape(B, 1)
    cb = coeff_b.reshape(B, 1)

    xt, nout = pl.pallas_call(
        _scale_body,
        grid=(B // BB,),
        in_specs=[
            pl.BlockSpec((BB, 1), lambda i: (i, 0)),
            pl.BlockSpec((BB, 1), lambda i: (i, 0)),
            pl.BlockSpec((BB, F), lambda i: (i, 0)),
            pl.BlockSpec((BB, F), lambda i: (i, 0)),
        ],
        out_specs=[
            pl.BlockSpec((BB, F), lambda i: (i, 0)),
            pl.BlockSpec((BB, F), lambda i: (i, 0)),
        ],
        out_shape=[
            jax.ShapeDtypeStruct((B, F), jnp.float32),
            jax.ShapeDtypeStruct((B, F), jnp.float32),
        ],
    )(ca, cb, x2, n2)

    return (xt.reshape(B, C, L), nout.reshape(B, C, L))


# speedup vs baseline: 1.0882x; 1.0882x over previous
import functools

import jax
import jax.numpy as jnp
from jax import lax
from jax.experimental import pallas as pl
from jax.experimental.pallas import tpu as pltpu
from jax.experimental.pallas import tpu_sc as plsc

_NC = 2
_NS = 16
_L = 16
_NW = _NC * _NS


def _sc_gather_coeffs(t, table_a, table_b):
    B = t.shape[0]
    T = table_a.shape[0]
    per_w = B // _NW

    mesh = plsc.VectorSubcoreMesh(core_axis_name="c", subcore_axis_name="s")

    @functools.partial(
        pl.kernel,
        out_type=[
            jax.ShapeDtypeStruct((B,), jnp.float32),
            jax.ShapeDtypeStruct((B,), jnp.float32),
        ],
        mesh=mesh,
        compiler_params=pltpu.CompilerParams(needs_layout_passes=False),
        scratch_types=[
            pltpu.VMEM((per_w,), jnp.int32),
            pltpu.VMEM((T,), jnp.float32),
            pltpu.VMEM((T,), jnp.float32),
            pltpu.VMEM((per_w,), jnp.float32),
            pltpu.VMEM((per_w,), jnp.float32),
        ],
    )
    def gather_kernel(t_hbm, a_hbm, b_hbm, out_a_hbm, out_b_hbm,
                      idx_v, a_v, b_v, oa_v, ob_v):
        wid = lax.axis_index("s") * _NC + lax.axis_index("c")
        base = wid * per_w
        pltpu.sync_copy(t_hbm.at[pl.ds(base, per_w)], idx_v)
        pltpu.sync_copy(a_hbm, a_v)
        pltpu.sync_copy(b_hbm, b_v)
        for i in range(per_w // _L):
            iv = idx_v[pl.ds(i * _L, _L)]
            oa_v[pl.ds(i * _L, _L)] = plsc.load_gather(a_v, [iv])
            ob_v[pl.ds(i * _L, _L)] = plsc.load_gather(b_v, [iv])
        pltpu.sync_copy(oa_v, out_a_hbm.at[pl.ds(base, per_w)])
        pltpu.sync_copy(ob_v, out_b_hbm.at[pl.ds(base, per_w)])

    return gather_kernel(t, table_a, table_b)


def _scale_body(a_ref, b_ref, x_ref, n_ref, xt_ref):
    xt_ref[...] = a_ref[...] * x_ref[...] + b_ref[...] * n_ref[...]


def kernel(x_0, t, sqrt_alphas_cumprod, sqrt_one_minus_alphas_cumprod, noise):
    B, C, L = x_0.shape
    F = C * L
    BB = 16

    coeff_a, coeff_b = _sc_gather_coeffs(
        t, sqrt_alphas_cumprod, sqrt_one_minus_alphas_cumprod)

    x2 = x_0.reshape(B, F)
    n2 = noise.reshape(B, F)
    ca = coeff_a.reshape(B, 1)
    cb = coeff_b.reshape(B, 1)

    xt = pl.pallas_call(
        _scale_body,
        grid=(B // BB,),
        in_specs=[
            pl.BlockSpec((BB, 1), lambda i: (i, 0)),
            pl.BlockSpec((BB, 1), lambda i: (i, 0)),
            pl.BlockSpec((BB, F), lambda i: (i, 0)),
            pl.BlockSpec((BB, F), lambda i: (i, 0)),
        ],
        out_specs=pl.BlockSpec((BB, F), lambda i: (i, 0)),
        out_shape=jax.ShapeDtypeStruct((B, F), jnp.float32),
    )(ca, cb, x2, n2)

    return (xt.reshape(B, C, L), noise)


# speedup vs baseline: 2.2540x; 2.0713x over previous
import functools

import jax
import jax.numpy as jnp
from jax import lax
from jax.experimental import pallas as pl
from jax.experimental.pallas import tpu as pltpu
from jax.experimental.pallas import tpu_sc as plsc

_NC = 2
_NS = 16
_L = 16
_NW = _NC * _NS


def _sc_gather_coeffs(t, table_a, table_b):
    B = t.shape[0]
    T = table_a.shape[0]
    per_w = B // _NW

    mesh = plsc.VectorSubcoreMesh(core_axis_name="c", subcore_axis_name="s")

    @functools.partial(
        pl.kernel,
        out_type=[
            jax.ShapeDtypeStruct((B,), jnp.float32),
            jax.ShapeDtypeStruct((B,), jnp.float32),
        ],
        mesh=mesh,
        compiler_params=pltpu.CompilerParams(needs_layout_passes=False),
        scratch_types=[
            pltpu.VMEM((per_w,), jnp.int32),
            pltpu.VMEM((T,), jnp.float32),
            pltpu.VMEM((T,), jnp.float32),
            pltpu.VMEM((per_w,), jnp.float32),
            pltpu.VMEM((per_w,), jnp.float32),
        ],
    )
    def gather_kernel(t_hbm, a_hbm, b_hbm, out_a_hbm, out_b_hbm,
                      idx_v, a_v, b_v, oa_v, ob_v):
        wid = lax.axis_index("s") * _NC + lax.axis_index("c")
        base = wid * per_w
        pltpu.sync_copy(t_hbm.at[pl.ds(base, per_w)], idx_v)
        pltpu.sync_copy(a_hbm, a_v)
        pltpu.sync_copy(b_hbm, b_v)
        for i in range(per_w // _L):
            iv = idx_v[pl.ds(i * _L, _L)]
            oa_v[pl.ds(i * _L, _L)] = plsc.load_gather(a_v, [iv])
            ob_v[pl.ds(i * _L, _L)] = plsc.load_gather(b_v, [iv])
        pltpu.sync_copy(oa_v, out_a_hbm.at[pl.ds(base, per_w)])
        pltpu.sync_copy(ob_v, out_b_hbm.at[pl.ds(base, per_w)])

    return gather_kernel(t, table_a, table_b)


def _scale_body(a_ref, b_ref, x_ref, n_ref, xt_ref):
    xt_ref[...] = a_ref[...] * x_ref[...] + b_ref[...] * n_ref[...]


def kernel(x_0, t, sqrt_alphas_cumprod, sqrt_one_minus_alphas_cumprod, noise):
    B, C, L = x_0.shape
    BB = 16

    coeff_a, coeff_b = _sc_gather_coeffs(
        t, sqrt_alphas_cumprod, sqrt_one_minus_alphas_cumprod)

    ca = coeff_a.reshape(B, 1, 1)
    cb = coeff_b.reshape(B, 1, 1)

    xt = pl.pallas_call(
        _scale_body,
        grid=(B // BB,),
        in_specs=[
            pl.BlockSpec((BB, 1, 1), lambda i: (i, 0, 0)),
            pl.BlockSpec((BB, 1, 1), lambda i: (i, 0, 0)),
            pl.BlockSpec((BB, C, L), lambda i: (i, 0, 0)),
            pl.BlockSpec((BB, C, L), lambda i: (i, 0, 0)),
        ],
        out_specs=pl.BlockSpec((BB, C, L), lambda i: (i, 0, 0)),
        out_shape=jax.ShapeDtypeStruct((B, C, L), jnp.float32),
    )(ca, cb, x_0, noise)

    return (xt, noise)


# speedup vs baseline: 2.5524x; 1.1324x over previous
import functools

import jax
import jax.numpy as jnp
from jax import lax
from jax.experimental import pallas as pl
from jax.experimental.pallas import tpu as pltpu
from jax.experimental.pallas import tpu_sc as plsc

_NC = 2
_NS = 16
_L = 16
_NW = _NC * _NS


def _sc_gather_coeffs(t, table_a, table_b):
    B = t.shape[0]
    T = table_a.shape[0]
    per_w = B // _NW

    mesh = plsc.VectorSubcoreMesh(core_axis_name="c", subcore_axis_name="s")

    @functools.partial(
        pl.kernel,
        out_type=[
            jax.ShapeDtypeStruct((B,), jnp.float32),
            jax.ShapeDtypeStruct((B,), jnp.float32),
        ],
        mesh=mesh,
        compiler_params=pltpu.CompilerParams(needs_layout_passes=False),
        scratch_types=[
            pltpu.VMEM((per_w,), jnp.int32),
            pltpu.VMEM((T,), jnp.float32),
            pltpu.VMEM((T,), jnp.float32),
            pltpu.VMEM((per_w,), jnp.float32),
            pltpu.VMEM((per_w,), jnp.float32),
        ],
    )
    def gather_kernel(t_hbm, a_hbm, b_hbm, out_a_hbm, out_b_hbm,
                      idx_v, a_v, b_v, oa_v, ob_v):
        wid = lax.axis_index("s") * _NC + lax.axis_index("c")
        base = wid * per_w
        pltpu.sync_copy(t_hbm.at[pl.ds(base, per_w)], idx_v)
        pltpu.sync_copy(a_hbm, a_v)
        pltpu.sync_copy(b_hbm, b_v)
        for i in range(per_w // _L):
            iv = idx_v[pl.ds(i * _L, _L)]
            oa_v[pl.ds(i * _L, _L)] = plsc.load_gather(a_v, [iv])
            ob_v[pl.ds(i * _L, _L)] = plsc.load_gather(b_v, [iv])
        pltpu.sync_copy(oa_v, out_a_hbm.at[pl.ds(base, per_w)])
        pltpu.sync_copy(ob_v, out_b_hbm.at[pl.ds(base, per_w)])

    return gather_kernel(t, table_a, table_b)


def _scale_body(a_ref, b_ref, x_ref, n_ref, xt_ref):
    xt_ref[...] = a_ref[...] * x_ref[...] + b_ref[...] * n_ref[...]


def kernel(x_0, t, sqrt_alphas_cumprod, sqrt_one_minus_alphas_cumprod, noise):
    B, C, L = x_0.shape
    BB = 64

    coeff_a, coeff_b = _sc_gather_coeffs(
        t, sqrt_alphas_cumprod, sqrt_one_minus_alphas_cumprod)

    ca = coeff_a.reshape(B, 1, 1)
    cb = coeff_b.reshape(B, 1, 1)

    xt = pl.pallas_call(
        _scale_body,
        grid=(B // BB,),
        in_specs=[
            pl.BlockSpec((BB, 1, 1), lambda i: (i, 0, 0)),
            pl.BlockSpec((BB, 1, 1), lambda i: (i, 0, 0)),
            pl.BlockSpec((BB, C, L), lambda i: (i, 0, 0)),
            pl.BlockSpec((BB, C, L), lambda i: (i, 0, 0)),
        ],
        out_specs=pl.BlockSpec((BB, C, L), lambda i: (i, 0, 0)),
        out_shape=jax.ShapeDtypeStruct((B, C, L), jnp.float32),
    )(ca, cb, x_0, noise)

    return (xt, noise)


# speedup vs baseline: 2.5533x; 1.0004x over previous
import functools

import jax
import jax.numpy as jnp
from jax import lax
from jax.experimental import pallas as pl
from jax.experimental.pallas import tpu as pltpu
from jax.experimental.pallas import tpu_sc as plsc

_NC = 2
_NS = 16
_L = 16
_NW = _NC * _NS


def _sc_gather_coeffs(t, table_a, table_b):
    B = t.shape[0]
    T = table_a.shape[0]
    per_w = B // _NW

    mesh = plsc.VectorSubcoreMesh(core_axis_name="c", subcore_axis_name="s")

    @functools.partial(
        pl.kernel,
        out_type=[
            jax.ShapeDtypeStruct((B,), jnp.float32),
            jax.ShapeDtypeStruct((B,), jnp.float32),
        ],
        mesh=mesh,
        compiler_params=pltpu.CompilerParams(needs_layout_passes=False),
        scratch_types=[
            pltpu.VMEM((per_w,), jnp.int32),
            pltpu.VMEM((T,), jnp.float32),
            pltpu.VMEM((T,), jnp.float32),
            pltpu.VMEM((per_w,), jnp.float32),
            pltpu.VMEM((per_w,), jnp.float32),
        ],
    )
    def gather_kernel(t_hbm, a_hbm, b_hbm, out_a_hbm, out_b_hbm,
                      idx_v, a_v, b_v, oa_v, ob_v):
        wid = lax.axis_index("s") * _NC + lax.axis_index("c")
        base = wid * per_w
        pltpu.sync_copy(t_hbm.at[pl.ds(base, per_w)], idx_v)
        pltpu.sync_copy(a_hbm, a_v)
        pltpu.sync_copy(b_hbm, b_v)
        for i in range(per_w // _L):
            iv = idx_v[pl.ds(i * _L, _L)]
            oa_v[pl.ds(i * _L, _L)] = plsc.load_gather(a_v, [iv])
            ob_v[pl.ds(i * _L, _L)] = plsc.load_gather(b_v, [iv])
        pltpu.sync_copy(oa_v, out_a_hbm.at[pl.ds(base, per_w)])
        pltpu.sync_copy(ob_v, out_b_hbm.at[pl.ds(base, per_w)])

    return gather_kernel(t, table_a, table_b)


def _scale_body(a_ref, b_ref, x_ref, n_ref, xt_ref):
    xt_ref[...] = a_ref[...] * x_ref[...] + b_ref[...] * n_ref[...]


def kernel(x_0, t, sqrt_alphas_cumprod, sqrt_one_minus_alphas_cumprod, noise):
    B, C, L = x_0.shape
    BB = 128

    coeff_a, coeff_b = _sc_gather_coeffs(
        t, sqrt_alphas_cumprod, sqrt_one_minus_alphas_cumprod)

    ca = coeff_a.reshape(B, 1, 1)
    cb = coeff_b.reshape(B, 1, 1)

    xt = pl.pallas_call(
        _scale_body,
        grid=(B // BB,),
        in_specs=[
            pl.BlockSpec((BB, 1, 1), lambda i: (i, 0, 0)),
            pl.BlockSpec((BB, 1, 1), lambda i: (i, 0, 0)),
            pl.BlockSpec((BB, C, L), lambda i: (i, 0, 0)),
            pl.BlockSpec((BB, C, L), lambda i: (i, 0, 0)),
        ],
        out_specs=pl.BlockSpec((BB, C, L), lambda i: (i, 0, 0)),
        out_shape=jax.ShapeDtypeStruct((B, C, L), jnp.float32),
        compiler_params=pltpu.CompilerParams(
            vmem_limit_bytes=100 * 1024 * 1024),
    )(ca, cb, x_0, noise)

    return (xt, noise)
